# baseline (speedup 1.0000x reference)
import jax
import jax.numpy as jnp
from jax.experimental import pallas as pl
from jax.experimental.pallas import tpu as pltpu

_KS = 30
_STR = 5
_LU = 6000
_P1_PAD = 1280
_FEAT = 234
_FEAT_PAD = 256
_K1 = 672
_NBLK1 = 10
_TB = 1024


def _band(w, shape):
    tap = (jax.lax.broadcasted_iota(jnp.int32, shape, 0)
           - _STR * jax.lax.broadcasted_iota(jnp.int32, shape, 1))
    acc = jnp.zeros(shape, jnp.float32)
    for t in range(_KS):
        acc = acc + jnp.where(tap == t, w[t], 0.0)
    return acc


def _net_kernel(b1_ref, b2_ref, w1_ref, w2_ref, x_ref, wout_ref, bout_ref,
                out_ref, B1, B2, woutT, p1):
    i = pl.program_id(0)

    @pl.when(i == 0)
    def _():
        B1[...] = _band(w1_ref, (_K1, 128)).astype(jnp.bfloat16)
        B2[...] = _band(w2_ref, (_K1, 128)).astype(jnp.bfloat16)
        woutT[...] = jnp.concatenate(
            [wout_ref[...].T,
             jnp.zeros((_FEAT_PAD - _FEAT, 5), jnp.float32)], axis=0)

    b1 = b1_ref[0]
    b2 = b2_ref[0]

    for j in range(_NBLK1):
        lo = 640 * j
        span = min(_K1, _LU - lo)
        acc = jnp.dot(x_ref[:, lo:lo + span], B1[0:span, :],
                      preferred_element_type=jnp.float32)
        p1[:, 128 * j:128 * (j + 1)] = jnp.maximum(
            acc + b1, 0.0).astype(jnp.bfloat16)

    acc2a = jnp.dot(p1[:, 0:_K1], B2[...],
                    preferred_element_type=jnp.float32)
    acc2b = jnp.dot(p1[:, 640:_P1_PAD], B2[0:_P1_PAD - 640, :],
                    preferred_element_type=jnp.float32)
    h = jnp.maximum(jnp.concatenate([acc2a, acc2b], axis=1) + b2, 0.0)

    out_ref[...] = (jnp.dot(h, woutT[...],
                            preferred_element_type=jnp.float32)
                    + bout_ref[...])


def _softmax_dim0_kernel(z_ref, o_ref):
    z = z_ref[...]
    m = jnp.max(z, axis=0, keepdims=True)
    e = jnp.exp(z - m)
    o_ref[...] = e / jnp.sum(e, axis=0, keepdims=True)


def kernel(x, w1, b1, w2, b2, wout, bout):
    n, ch, length = x.shape
    assert ch == 1 and length >= _LU

    tb = n if n <= _TB else _TB
    assert tb % 8 == 0
    n_pad = ((n + tb - 1) // tb) * tb
    x2 = x.reshape(n, length)[:, :_LU].astype(jnp.bfloat16)
    if n_pad != n:
        x2 = jnp.pad(x2, ((0, n_pad - n), (0, 0)))

    logits_pad = pl.pallas_call(
        _net_kernel,
        out_shape=jax.ShapeDtypeStruct((n_pad, 5), jnp.float32),
        grid=(n_pad // tb,),
        in_specs=[
            pl.BlockSpec(memory_space=pltpu.MemorySpace.SMEM),
            pl.BlockSpec(memory_space=pltpu.MemorySpace.SMEM),
            pl.BlockSpec(memory_space=pltpu.MemorySpace.SMEM),
            pl.BlockSpec(memory_space=pltpu.MemorySpace.SMEM),
            pl.BlockSpec((tb, _LU), lambda i: (i, 0)),
            pl.BlockSpec(memory_space=pltpu.MemorySpace.VMEM),
            pl.BlockSpec(memory_space=pltpu.MemorySpace.VMEM),
        ],
        out_specs=pl.BlockSpec((tb, 5), lambda i: (i, 0)),
        scratch_shapes=[
            pltpu.VMEM((_K1, 128), jnp.bfloat16),
            pltpu.VMEM((_K1, 128), jnp.bfloat16),
            pltpu.VMEM((_FEAT_PAD, 5), jnp.float32),
            pltpu.VMEM((tb, _P1_PAD), jnp.bfloat16),
        ],
        compiler_params=pltpu.CompilerParams(
            dimension_semantics=("arbitrary",),
            vmem_limit_bytes=48 * 1024 * 1024),
    )(b1.astype(jnp.float32), b2.astype(jnp.float32),
      w1.astype(jnp.float32), w2.astype(jnp.float32),
      x2, wout.astype(jnp.float32), bout.reshape(1, 5).astype(jnp.float32))

    logits = logits_pad[:n]

    probs = pl.pallas_call(
        _softmax_dim0_kernel,
        out_shape=jax.ShapeDtypeStruct((n, 5), jnp.float32),
        in_specs=[pl.BlockSpec(memory_space=pltpu.MemorySpace.VMEM)],
        out_specs=pl.BlockSpec(memory_space=pltpu.MemorySpace.VMEM),
    )(logits)

    return probs.reshape(n, 1, 5)


# speedup vs baseline: 1.0127x; 1.0127x over previous
import jax
import jax.numpy as jnp
from jax.experimental import pallas as pl
from jax.experimental.pallas import tpu as pltpu

_KS = 30
_STR = 5
_LU = 6000
_P1_PAD = 1280
_FEAT = 234
_FEAT_PAD = 256
_K1 = 672
_NBLK1 = 10
_TB = 512


def _band(w, shape):
    tap = (jax.lax.broadcasted_iota(jnp.int32, shape, 0)
           - _STR * jax.lax.broadcasted_iota(jnp.int32, shape, 1))
    acc = jnp.zeros(shape, jnp.float32)
    for t in range(_KS):
        acc = acc + jnp.where(tap == t, w[t], 0.0)
    return acc


def _net_kernel(b1_ref, b2_ref, w1_ref, w2_ref, x_ref, wout_ref, bout_ref,
                out_ref, B1, B2, woutT, p1):
    i = pl.program_id(0)

    @pl.when(i == 0)
    def _():
        B1[...] = _band(w1_ref, (_K1, 128)).astype(jnp.bfloat16)
        B2[...] = _band(w2_ref, (_K1, 128)).astype(jnp.bfloat16)
        woutT[...] = jnp.concatenate(
            [wout_ref[...].T,
             jnp.zeros((_FEAT_PAD - _FEAT, 5), jnp.float32)], axis=0)

    b1 = b1_ref[0]
    b2 = b2_ref[0]

    for j in range(_NBLK1):
        lo = 640 * j
        span = min(_K1, _LU - lo)
        acc = jnp.dot(x_ref[:, lo:lo + span], B1[0:span, :],
                      preferred_element_type=jnp.float32)
        p1[:, 128 * j:128 * (j + 1)] = jnp.maximum(
            acc + b1, 0.0).astype(jnp.bfloat16)

    acc2a = jnp.dot(p1[:, 0:_K1], B2[...],
                    preferred_element_type=jnp.float32)
    acc2b = jnp.dot(p1[:, 640:_P1_PAD], B2[0:_P1_PAD - 640, :],
                    preferred_element_type=jnp.float32)
    h = jnp.maximum(jnp.concatenate([acc2a, acc2b], axis=1) + b2, 0.0)

    out_ref[...] = (jnp.dot(h, woutT[...],
                            preferred_element_type=jnp.float32)
                    + bout_ref[...])


def _softmax_dim0_kernel(z_ref, o_ref):
    z = z_ref[...]
    m = jnp.max(z, axis=0, keepdims=True)
    e = jnp.exp(z - m)
    o_ref[...] = e / jnp.sum(e, axis=0, keepdims=True)


def kernel(x, w1, b1, w2, b2, wout, bout):
    n, ch, length = x.shape
    assert ch == 1 and length >= _LU

    tb = n if n <= _TB else _TB
    assert tb % 8 == 0
    n_pad = ((n + tb - 1) // tb) * tb
    x2 = x.reshape(n, length)[:, :_LU].astype(jnp.bfloat16)
    if n_pad != n:
        x2 = jnp.pad(x2, ((0, n_pad - n), (0, 0)))

    logits_pad = pl.pallas_call(
        _net_kernel,
        out_shape=jax.ShapeDtypeStruct((n_pad, 5), jnp.float32),
        grid=(n_pad // tb,),
        in_specs=[
            pl.BlockSpec(memory_space=pltpu.MemorySpace.SMEM),
            pl.BlockSpec(memory_space=pltpu.MemorySpace.SMEM),
            pl.BlockSpec(memory_space=pltpu.MemorySpace.SMEM),
            pl.BlockSpec(memory_space=pltpu.MemorySpace.SMEM),
            pl.BlockSpec((tb, _LU), lambda i: (i, 0)),
            pl.BlockSpec(memory_space=pltpu.MemorySpace.VMEM),
            pl.BlockSpec(memory_space=pltpu.MemorySpace.VMEM),
        ],
        out_specs=pl.BlockSpec((tb, 5), lambda i: (i, 0)),
        scratch_shapes=[
            pltpu.VMEM((_K1, 128), jnp.bfloat16),
            pltpu.VMEM((_K1, 128), jnp.bfloat16),
            pltpu.VMEM((_FEAT_PAD, 5), jnp.float32),
            pltpu.VMEM((tb, _P1_PAD), jnp.bfloat16),
        ],
        compiler_params=pltpu.CompilerParams(
            dimension_semantics=("arbitrary",),
            vmem_limit_bytes=48 * 1024 * 1024),
    )(b1.astype(jnp.float32), b2.astype(jnp.float32),
      w1.astype(jnp.float32), w2.astype(jnp.float32),
      x2, wout.astype(jnp.float32), bout.reshape(1, 5).astype(jnp.float32))

    logits = logits_pad[:n]

    probs = pl.pallas_call(
        _softmax_dim0_kernel,
        out_shape=jax.ShapeDtypeStruct((n, 5), jnp.float32),
        in_specs=[pl.BlockSpec(memory_space=pltpu.MemorySpace.VMEM)],
        out_specs=pl.BlockSpec(memory_space=pltpu.MemorySpace.VMEM),
    )(logits)

    return probs.reshape(n, 1, 5)


# speedup vs baseline: 1.0991x; 1.0854x over previous
import functools

import jax
import jax.numpy as jnp
from jax.experimental import pallas as pl
from jax.experimental.pallas import tpu as pltpu

_KS = 30
_STR = 5
_LU = 6000
_P1_PAD = 1280
_FEAT = 234
_FEAT_PAD = 256
_K1 = 672
_NBLK1 = 10
_TB = 512


def _band(w, shape):
    tap = (jax.lax.broadcasted_iota(jnp.int32, shape, 0)
           - _STR * jax.lax.broadcasted_iota(jnp.int32, shape, 1))
    acc = jnp.zeros(shape, jnp.float32)
    for t in range(_KS):
        acc = acc + jnp.where(tap == t, w[t], 0.0)
    return acc


def _net_kernel(b1_ref, b2_ref, w1_ref, w2_ref, x_ref, wout_ref, bout_ref,
                out_ref, B1, B2, woutT, p1, *, fuse_softmax):
    i = pl.program_id(0)

    @pl.when(i == 0)
    def _():
        B1[...] = _band(w1_ref, (_K1, 128)).astype(jnp.bfloat16)
        B2[...] = _band(w2_ref, (_K1, 128)).astype(jnp.bfloat16)
        woutT[...] = jnp.concatenate(
            [wout_ref[...].T,
             jnp.zeros((_FEAT_PAD - _FEAT, 5), jnp.float32)], axis=0)

    b1 = b1_ref[0]
    b2 = b2_ref[0]

    for j in range(_NBLK1):
        lo = 640 * j
        span = min(_K1, _LU - lo)
        acc = jnp.dot(x_ref[:, lo:lo + span], B1[0:span, :],
                      preferred_element_type=jnp.float32)
        p1[:, 128 * j:128 * (j + 1)] = jnp.maximum(
            acc + b1, 0.0).astype(jnp.bfloat16)

    acc2a = jnp.dot(p1[:, 0:_K1], B2[...],
                    preferred_element_type=jnp.float32)
    acc2b = jnp.dot(p1[:, 640:_P1_PAD], B2[0:_P1_PAD - 640, :],
                    preferred_element_type=jnp.float32)
    h = jnp.maximum(jnp.concatenate([acc2a, acc2b], axis=1) + b2, 0.0)

    tb = x_ref.shape[0]
    row0 = pl.multiple_of(i * tb, tb)
    out_ref[pl.ds(row0, tb), :] = (
        jnp.dot(h, woutT[...], preferred_element_type=jnp.float32)
        + bout_ref[...])

    if fuse_softmax:
        @pl.when(i == pl.num_programs(0) - 1)
        def _():
            z = out_ref[...]
            m = jnp.max(z, axis=0, keepdims=True)
            e = jnp.exp(z - m)
            out_ref[...] = e / jnp.sum(e, axis=0, keepdims=True)


def _softmax_dim0_kernel(z_ref, o_ref):
    z = z_ref[...]
    m = jnp.max(z, axis=0, keepdims=True)
    e = jnp.exp(z - m)
    o_ref[...] = e / jnp.sum(e, axis=0, keepdims=True)


def kernel(x, w1, b1, w2, b2, wout, bout):
    n, ch, length = x.shape
    assert ch == 1 and length >= _LU

    tb = n if n <= _TB else _TB
    assert tb % 8 == 0
    n_pad = ((n + tb - 1) // tb) * tb
    x2 = x.reshape(n, length)[:, :_LU].astype(jnp.bfloat16)
    if n_pad != n:
        x2 = jnp.pad(x2, ((0, n_pad - n), (0, 0)))

    fuse_softmax = (n_pad == n)
    out_pad = pl.pallas_call(
        functools.partial(_net_kernel, fuse_softmax=fuse_softmax),
        out_shape=jax.ShapeDtypeStruct((n_pad, 5), jnp.float32),
        grid=(n_pad // tb,),
        in_specs=[
            pl.BlockSpec(memory_space=pltpu.MemorySpace.SMEM),
            pl.BlockSpec(memory_space=pltpu.MemorySpace.SMEM),
            pl.BlockSpec(memory_space=pltpu.MemorySpace.SMEM),
            pl.BlockSpec(memory_space=pltpu.MemorySpace.SMEM),
            pl.BlockSpec((tb, _LU), lambda i: (i, 0)),
            pl.BlockSpec(memory_space=pltpu.MemorySpace.VMEM),
            pl.BlockSpec(memory_space=pltpu.MemorySpace.VMEM),
        ],
        out_specs=pl.BlockSpec((n_pad, 5), lambda i: (0, 0)),
        scratch_shapes=[
            pltpu.VMEM((_K1, 128), jnp.bfloat16),
            pltpu.VMEM((_K1, 128), jnp.bfloat16),
            pltpu.VMEM((_FEAT_PAD, 5), jnp.float32),
            pltpu.VMEM((tb, _P1_PAD), jnp.bfloat16),
        ],
        compiler_params=pltpu.CompilerParams(
            dimension_semantics=("arbitrary",),
            vmem_limit_bytes=48 * 1024 * 1024),
    )(b1.astype(jnp.float32), b2.astype(jnp.float32),
      w1.astype(jnp.float32), w2.astype(jnp.float32),
      x2, wout.astype(jnp.float32), bout.reshape(1, 5).astype(jnp.float32))

    if fuse_softmax:
        probs = out_pad
    else:
        probs = pl.pallas_call(
            _softmax_dim0_kernel,
            out_shape=jax.ShapeDtypeStruct((n, 5), jnp.float32),
            in_specs=[pl.BlockSpec(memory_space=pltpu.MemorySpace.VMEM)],
            out_specs=pl.BlockSpec(memory_space=pltpu.MemorySpace.VMEM),
        )(out_pad[:n])

    return probs.reshape(n, 1, 5)
